# Initial kernel scaffold; baseline (speedup 1.0000x reference)
import jax
import jax.numpy as jnp
from jax.experimental import pallas as pl


def kernel(x, v, Wq, Wk, Wu, W1, b1, W2, b2, W3, b3):
    raise NotImplementedError("write your pallas kernel here")


# speedup vs baseline: 7.2148x; 7.2148x over previous
import jax
import jax.numpy as jnp
from jax.experimental import pallas as pl

B, N, C, KNN, H = 2, 2048, 64, 16, 32
R = 256
F32 = jnp.float32
D3 = 3 * C


def _proj_kernel(vt_ref, wq_ref, wk_ref, wu_ref, qf_ref, kf_ref, uf_ref, kn_ref):
    vt = vt_ref[0]
    qf = jnp.dot(vt, wq_ref[...], preferred_element_type=F32)
    kf = jnp.dot(vt, wk_ref[...], preferred_element_type=F32)
    uf = jnp.dot(vt, wu_ref[...], preferred_element_type=F32)
    qf_ref[0] = qf
    kf_ref[0] = kf
    uf_ref[0] = uf
    ks = kf * kf
    s = ks[:, 0:C] + ks[:, C:2 * C] + ks[:, 2 * C:3 * C]
    kn = jnp.sum(jnp.sqrt(s), axis=1, keepdims=True)
    kn_ref[0] = kn.T


def _main_kernel(x_ref, xt_ref, xb_ref, xtb_ref, qf_ref, kf_ref, uf_ref,
                 kn_ref, w1_ref, b1_ref, w2_ref, b2_ref, w3_ref, b3_ref,
                 out_ref):
    rb = pl.program_id(1)
    xr = x_ref[0]
    xa = xt_ref[0]
    xrb = xb_ref[0]
    xab = xtb_ref[0]
    qf = qf_ref[0]
    kf = kf_ref[0]
    uf = uf_ref[0]
    kn = kn_ref[0]

    sq_r = jnp.sum(xr * xr, axis=1, keepdims=True)
    sq_a = jnp.sum(xa * xa, axis=0, keepdims=True)
    g = jax.lax.dot_general(xrb, xab, (((1,), (0,)), ((), ())),
                            preferred_element_type=F32)
    d2 = sq_r + sq_a - 2.0 * g

    colids = jax.lax.broadcasted_iota(jnp.int32, (R, N), 1)
    rowg = rb * R + jax.lax.broadcasted_iota(jnp.int32, (R, N), 0)
    d2 = jnp.where(colids == rowg, jnp.inf, d2)

    dotm = jax.lax.dot_general(qf, kf, (((1,), (1,)), ((), ())),
                               preferred_element_type=F32,
                               precision=jax.lax.Precision.HIGHEST)

    qs = qf * qf
    s = qs[:, 0:C] + qs[:, C:2 * C] + qs[:, 2 * C:3 * C]
    qn = jnp.sum(jnp.sqrt(s), axis=1, keepdims=True)

    knb = jnp.broadcast_to(kn, (R, N))
    xa0 = jnp.broadcast_to(xa[0:1, :], (R, N))
    xa1 = jnp.broadcast_to(xa[1:2, :], (R, N))
    xa2 = jnp.broadcast_to(xa[2:3, :], (R, N))
    d2w = d2
    dist_l, dot_l, kn_l, idx_l = [], [], [], []
    for _ in range(KNN):
        m = jnp.min(d2w, axis=1, keepdims=True)
        ismin = d2w == m
        idxk = jnp.min(jnp.where(ismin, colids, N), axis=1, keepdims=True)
        oneh = colids == idxk
        xm0 = jnp.sum(jnp.where(oneh, xa0, 0.0), axis=1, keepdims=True)
        xm1 = jnp.sum(jnp.where(oneh, xa1, 0.0), axis=1, keepdims=True)
        xm2 = jnp.sum(jnp.where(oneh, xa2, 0.0), axis=1, keepdims=True)
        e0 = xr[:, 0:1] - xm0
        e1 = xr[:, 1:2] - xm1
        e2 = xr[:, 2:3] - xm2
        dist_l.append(jnp.sqrt(e0 * e0 + e1 * e1 + e2 * e2))
        dot_l.append(jnp.sum(jnp.where(oneh, dotm, 0.0), axis=1, keepdims=True))
        kn_l.append(jnp.sum(jnp.where(oneh, knb, 0.0), axis=1, keepdims=True))
        idx_l.append(idxk)
        d2w = jnp.where(oneh, jnp.inf, d2w)

    dist = jnp.concatenate(dist_l, axis=1)
    dotf = jnp.concatenate(dot_l, axis=1)
    knf = jnp.concatenate(kn_l, axis=1)
    qnf = jnp.broadcast_to(qn, (R, KNN))

    bf = lambda t: t.astype(jnp.bfloat16)
    e = jnp.concatenate([qnf, knf, dotf, dist], axis=1)
    h1 = jax.nn.silu(jnp.dot(bf(e), w1_ref[...],
                             preferred_element_type=F32) + b1_ref[...])
    h2 = jax.nn.silu(jnp.dot(bf(h1), w2_ref[...],
                             preferred_element_type=F32) + b2_ref[...])
    lg = jnp.dot(bf(h2), w3_ref[...],
                 preferred_element_type=F32) + b3_ref[...]

    mx = jnp.max(lg, axis=1, keepdims=True)
    ex = jnp.exp(lg - mx)
    attn = ex / jnp.sum(ex, axis=1, keepdims=True)

    a = jnp.zeros((R, N), dtype=F32)
    for k in range(KNN):
        oneh = colids == idx_l[k]
        a = a + jnp.where(oneh, jnp.broadcast_to(attn[:, k:k + 1], (R, N)), 0.0)

    msg = jax.lax.dot_general(a, uf, (((1,), (0,)), ((), ())),
                              preferred_element_type=F32,
                              precision=jax.lax.Precision.HIGHEST)
    out_ref[0] = qf + msg


def kernel(x, v, Wq, Wk, Wu, W1, b1, W2, b2, W3, b3):
    bf = jnp.bfloat16
    vt = v.transpose(0, 1, 3, 2).reshape(B, N, D3).astype(bf)
    xt = x.transpose(0, 2, 1)
    xb = x.astype(bf)
    xtb = xt.astype(bf)

    eye3 = jnp.eye(3, dtype=F32)
    wqe = jnp.kron(eye3, Wq.T).astype(bf)
    wke = jnp.kron(eye3, Wk.T).astype(bf)
    wue = jnp.kron(eye3, Wu.T).astype(bf)

    i16 = jnp.eye(KNN, dtype=F32)
    w1e = jnp.kron(W1.T, i16).astype(bf)
    w2e = jnp.kron(W2.T, i16).astype(bf)
    w3e = jnp.kron(W3.T, i16).astype(bf)
    ones_k = jnp.ones((KNN,), F32)
    b1e = jnp.kron(b1, ones_k)[None]
    b2e = jnp.kron(b2, ones_k)[None]
    b3e = (b3[0] * ones_k)[None]

    nb = N // R
    qf, kf, uf, kn = pl.pallas_call(
        _proj_kernel,
        grid=(B, nb),
        in_specs=[
            pl.BlockSpec((1, R, D3), lambda b, r: (b, r, 0)),
            pl.BlockSpec((D3, D3), lambda b, r: (0, 0)),
            pl.BlockSpec((D3, D3), lambda b, r: (0, 0)),
            pl.BlockSpec((D3, D3), lambda b, r: (0, 0)),
        ],
        out_specs=[
            pl.BlockSpec((1, R, D3), lambda b, r: (b, r, 0)),
            pl.BlockSpec((1, R, D3), lambda b, r: (b, r, 0)),
            pl.BlockSpec((1, R, D3), lambda b, r: (b, r, 0)),
            pl.BlockSpec((1, 1, R), lambda b, r: (b, 0, r)),
        ],
        out_shape=[
            jax.ShapeDtypeStruct((B, N, D3), F32),
            jax.ShapeDtypeStruct((B, N, D3), F32),
            jax.ShapeDtypeStruct((B, N, D3), F32),
            jax.ShapeDtypeStruct((B, 1, N), F32),
        ],
    )(vt, wqe, wke, wue)

    out = pl.pallas_call(
        _main_kernel,
        grid=(B, nb),
        in_specs=[
            pl.BlockSpec((1, R, 3), lambda b, r: (b, r, 0)),
            pl.BlockSpec((1, 3, N), lambda b, r: (b, 0, 0)),
            pl.BlockSpec((1, R, 3), lambda b, r: (b, r, 0)),
            pl.BlockSpec((1, 3, N), lambda b, r: (b, 0, 0)),
            pl.BlockSpec((1, R, D3), lambda b, r: (b, r, 0)),
            pl.BlockSpec((1, N, D3), lambda b, r: (b, 0, 0)),
            pl.BlockSpec((1, N, D3), lambda b, r: (b, 0, 0)),
            pl.BlockSpec((1, 1, N), lambda b, r: (b, 0, 0)),
            pl.BlockSpec((4 * KNN, H * KNN), lambda b, r: (0, 0)),
            pl.BlockSpec((1, H * KNN), lambda b, r: (0, 0)),
            pl.BlockSpec((H * KNN, H * KNN), lambda b, r: (0, 0)),
            pl.BlockSpec((1, H * KNN), lambda b, r: (0, 0)),
            pl.BlockSpec((H * KNN, KNN), lambda b, r: (0, 0)),
            pl.BlockSpec((1, KNN), lambda b, r: (0, 0)),
        ],
        out_specs=pl.BlockSpec((1, R, D3), lambda b, r: (b, r, 0)),
        out_shape=jax.ShapeDtypeStruct((B, N, D3), F32),
    )(x, xt, xb, xtb, qf, kf, uf, kn, w1e, b1e, w2e, b2e, w3e, b3e)

    return out.reshape(B, N, 3, C).transpose(0, 1, 3, 2)
